# baseline (speedup 1.0000x reference)
import functools

import jax
import jax.numpy as jnp
from jax import lax
from jax.experimental import pallas as pl
from jax.experimental.pallas import tpu as pltpu
from jax.experimental.pallas import tpu_sc as plsc

B = 4096
NSTEP = 16
VPS = B // (NSTEP * 16)

_PAIRS = ((0, 1), (0, 2), (0, 3), (1, 2), (1, 3), (2, 3))


def _body(out_hbm, lab_hbm, sen_hbm, ad1_hbm, ad2_hbm, a0_hbm, a1_hbm,
          res_hbm,
          x_v, lab_v, g1_v, g2_v, ad1_v, ad2_v, a0_v, a1_v, res_v, sem):
    c = lax.axis_index("c")
    s = lax.axis_index("s")

    @pl.when((c == 0) & (s == 0))
    def _compute():
        copies = [
            pltpu.async_copy(out_hbm, x_v, sem),
            pltpu.async_copy(lab_hbm, lab_v, sem),
            pltpu.async_copy(sen_hbm.at[0], g1_v, sem),
            pltpu.async_copy(sen_hbm.at[1], g2_v, sem),
            pltpu.async_copy(ad1_hbm, ad1_v, sem),
            pltpu.async_copy(ad2_hbm, ad2_v, sem),
            pltpu.async_copy(a0_hbm, a0_v, sem),
            pltpu.async_copy(a1_hbm, a1_v, sem),
        ]
        for cp in copies:
            cp.wait()

        zero16 = jnp.zeros((16,), jnp.float32)
        ones16 = jnp.ones((16,), jnp.float32)
        a0 = a0_v[:]
        a1 = a1_v[:]

        def step(i, carry):
            accs = list(carry)
            base = i * (VPS * 16)
            for j in range(VPS):
                dsl = pl.ds(base + j * 16, 16)
                x = x_v[dsl]
                lab = lab_v[dsl]
                g1 = g1_v[dsl]
                g2 = g2_v[dsl]
                u = jnp.exp(-jnp.abs(x))
                z = u / (u + 2.0)
                z2 = z * z
                p = z2 * (1.0 / 9.0) + (1.0 / 7.0)
                p = p * z2 + (1.0 / 5.0)
                p = p * z2 + (1.0 / 3.0)
                p = p * z2 + 1.0
                ls = jnp.minimum(x, 0.0) - 2.0 * z * p
                dom = (ad1_v[dsl] == a0) & (ad2_v[dsl] == a1)
                binv = lab * 4 + g1 * 2 + g2
                for b in range(8):
                    m = dom & (binv == b)
                    accs[b] = accs[b] + jnp.where(m, ls, zero16)
                    accs[b + 8] = accs[b + 8] + jnp.where(m, ones16, zero16)
                accs[16] = accs[16] + lab.astype(jnp.float32)
            return tuple(accs)

        init = tuple([zero16] * 17)
        accs = lax.fori_loop(0, NSTEP, step, init)

        totals = [jnp.full((16,), jnp.sum(a)) for a in accs]
        means = [totals[b] / jnp.maximum(totals[b + 8], ones16)
                 for b in range(8)]
        pres = [jnp.where(totals[b + 8] > 0.0, ones16, zero16)
                for b in range(8)]
        labtot = totals[16]
        has = [jnp.where(labtot < float(B), ones16, zero16),
               jnp.where(labtot > 0.0, ones16, zero16)]
        res = zero16
        for l in range(2):
            gap = zero16
            for (i, j) in _PAIRS:
                w = pres[4 * l + i] * pres[4 * l + j]
                d = means[4 * l + i] - means[4 * l + j]
                gap = gap + w * d * d
            res = res + has[l] * gap
        res_v[:] = res
        pltpu.sync_copy(res_v, res_hbm)


@jax.jit
def _mean_loss_sc(outputs, labels, sen_groups, ad1, ad2, a0, a1):
    kfn = pl.kernel(
        _body,
        out_type=jax.ShapeDtypeStruct((16,), jnp.float32),
        mesh=plsc.VectorSubcoreMesh(core_axis_name="c", subcore_axis_name="s"),
        compiler_params=pltpu.CompilerParams(needs_layout_passes=False),
        scratch_types=[
            pltpu.VMEM((B,), jnp.float32),
            pltpu.VMEM((B,), jnp.int32),
            pltpu.VMEM((B,), jnp.int32),
            pltpu.VMEM((B,), jnp.int32),
            pltpu.VMEM((B,), jnp.int32),
            pltpu.VMEM((B,), jnp.int32),
            pltpu.VMEM((16,), jnp.int32),
            pltpu.VMEM((16,), jnp.int32),
            pltpu.VMEM((16,), jnp.float32),
            pltpu.SemaphoreType.DMA,
        ],
    )
    return kfn(outputs, labels, sen_groups, ad1, ad2, a0, a1)


def kernel(outputs, labels, sen_group_name, sen_groups, ad1, ad2, a_map):
    a0 = jnp.full((16,), a_map[0, 0], jnp.int32)
    a1 = jnp.full((16,), a_map[0, 1], jnp.int32)
    out = _mean_loss_sc(outputs, labels, sen_groups, ad1, ad2, a0, a1)
    return out[:1]
